# baseline (speedup 1.0000x reference)
import jax
import jax.numpy as jnp
from jax.experimental import pallas as pl
from jax.experimental.pallas import tpu as pltpu

N_ROWS = 18432
N_STATES = 1024
Z_DIM = 64
BLOCK = 1024
N_BLOCKS = N_ROWS // BLOCK
N_GRID = N_BLOCKS + 1
COMMITMENT_COST = 0.25


def _vq_kernel(x_mm_ref, x_q_ref, w_ref,
               loss_ref, q_ref, perp_ref, enc_ref,
               mm2_ref, rn_ref, wn_ref, iota_ref, counts_ref, sse_ref):
    s = pl.program_id(0)
    w = w_ref[...]

    @pl.when(s == 0)
    def _init():
        wn_ref[...] = jnp.sum(w * w, axis=1).reshape(1, N_STATES)
        iota_ref[...] = jax.lax.broadcasted_iota(
            jnp.int32, (1, N_STATES), 1).astype(jnp.float32)
        counts_ref[...] = jnp.zeros_like(counts_ref)
        sse_ref[...] = jnp.zeros_like(sse_ref)
        mm2_ref[...] = jnp.zeros_like(mm2_ref)
        rn_ref[...] = jnp.zeros_like(rn_ref)

    d = rn_ref[...] + wn_ref[...] - mm2_ref[...]
    m = jnp.min(d, axis=1, keepdims=True)
    ii = iota_ref[...]
    idx = jnp.min(jnp.where(d == m, ii, jnp.float32(N_STATES)),
                  axis=1, keepdims=True)
    onehot = (ii == idx).astype(jnp.float32)
    enc_ref[...] = onehot

    xq = x_q_ref[...]
    q = jax.lax.dot_general(onehot, w, (((1,), (0,)), ((), ())),
                            preferred_element_type=jnp.float32)
    dq = q - xq
    q_ref[...] = xq + dq

    live = s >= 1
    ones_row = jnp.ones((1, BLOCK), jnp.float32)
    counts_ref[...] += jnp.where(
        live,
        jax.lax.dot_general(ones_row, onehot, (((1,), (0,)), ((), ())),
                            preferred_element_type=jnp.float32),
        0.0)
    sse_ref[...] += jnp.where(live, jnp.sum(dq * dq, keepdims=True), 0.0)

    x1 = x_mm_ref[...]
    rn_ref[...] = jnp.sum(x1 * x1, axis=1, keepdims=True)
    mm2_ref[...] = jax.lax.dot_general(
        x1 + x1, w, (((1,), (1,)), ((), ())),
        preferred_element_type=jnp.float32)

    @pl.when(s == N_GRID - 1)
    def _fini():
        sse = sse_ref[0, 0]
        loss_ref[...] = jnp.full((1, 1), (1.0 + COMMITMENT_COST)
                                 * sse / (N_ROWS * Z_DIM))
        avg = counts_ref[...] / N_ROWS
        ent = jnp.sum(avg * jnp.log(avg + 1e-10), keepdims=True)
        perp_ref[...] = jnp.exp(-ent)


@jax.jit
def kernel(inputs, weight):
    last = N_BLOCKS - 1
    loss, quantized_st, perp, encodings = pl.pallas_call(
        _vq_kernel,
        grid=(N_GRID,),
        in_specs=[
            pl.BlockSpec((BLOCK, Z_DIM),
                         lambda s: (jnp.minimum(s, last), 0)),
            pl.BlockSpec((BLOCK, Z_DIM),
                         lambda s: (jnp.clip(s - 1, 0, last), 0)),
            pl.BlockSpec((N_STATES, Z_DIM), lambda s: (0, 0)),
        ],
        out_specs=[
            pl.BlockSpec((1, 1), lambda s: (0, 0)),
            pl.BlockSpec((BLOCK, Z_DIM),
                         lambda s: (jnp.clip(s - 1, 0, last), 0)),
            pl.BlockSpec((1, 1), lambda s: (0, 0)),
            pl.BlockSpec((BLOCK, N_STATES),
                         lambda s: (jnp.clip(s - 1, 0, last), 0)),
        ],
        out_shape=[
            jax.ShapeDtypeStruct((1, 1), jnp.float32),
            jax.ShapeDtypeStruct((N_ROWS, Z_DIM), jnp.float32),
            jax.ShapeDtypeStruct((1, 1), jnp.float32),
            jax.ShapeDtypeStruct((N_ROWS, N_STATES), jnp.float32),
        ],
        scratch_shapes=[
            pltpu.VMEM((BLOCK, N_STATES), jnp.float32),
            pltpu.VMEM((BLOCK, 1), jnp.float32),
            pltpu.VMEM((1, N_STATES), jnp.float32),
            pltpu.VMEM((1, N_STATES), jnp.float32),
            pltpu.VMEM((1, N_STATES), jnp.float32),
            pltpu.VMEM((1, 1), jnp.float32),
        ],
    )(inputs, inputs, weight)
    return (loss.reshape(()), quantized_st, perp.reshape(()), encodings)
